# Initial kernel scaffold; baseline (speedup 1.0000x reference)
import jax
import jax.numpy as jnp
from jax.experimental import pallas as pl


def kernel(xyz_canon, scales):
    raise NotImplementedError("write your pallas kernel here")


# speedup vs baseline: 5.4393x; 5.4393x over previous
import functools

import jax
import jax.numpy as jnp
from jax import lax
from jax.experimental import pallas as pl
from jax.experimental.pallas import tpu as pltpu
from jax.experimental.pallas import tpu_sc as plsc

N = 10000
NUM_WORKERS = 32
ROWS_PER_WORKER = 313
NPAD = NUM_WORKERS * ROWS_PER_WORKER
NBLK = N // 16
COORD_LEN = NPAD + 16
PAD_COORD = 1.0e5


def _sc_topk(xs, ys, zs):
    mesh = plsc.VectorSubcoreMesh(
        core_axis_name="c", subcore_axis_name="s",
        num_cores=2, num_subcores=16,
    )

    @functools.partial(
        pl.kernel,
        out_type=jax.ShapeDtypeStruct((NPAD * 16,), jnp.float32),
        mesh=mesh,
        scratch_types=[
            pltpu.VMEM((COORD_LEN,), jnp.float32),
            pltpu.VMEM((COORD_LEN,), jnp.float32),
            pltpu.VMEM((COORD_LEN,), jnp.float32),
            pltpu.VMEM((ROWS_PER_WORKER * 16,), jnp.float32),
        ],
        compiler_params=pltpu.CompilerParams(needs_layout_passes=False),
    )
    def topk_kernel(xs_hbm, ys_hbm, zs_hbm, out_hbm, xv, yv, zv, res):
        wid = lax.axis_index("s") * 2 + lax.axis_index("c")
        pltpu.sync_copy(xs_hbm, xv)
        pltpu.sync_copy(ys_hbm, yv)
        pltpu.sync_copy(zs_hbm, zv)
        base_row = wid * ROWS_PER_WORKER

        def row_body(r, carry):
            row = base_row + r
            xi = xv[pl.ds(row, 16)][0]
            yi = yv[pl.ds(row, 16)][0]
            zi = zv[pl.ds(row, 16)][0]

            def blk_body(b, best):
                off = b * 16
                dx = xv[pl.ds(off, 16)] - xi
                dy = yv[pl.ds(off, 16)] - yi
                dz = zv[pl.ds(off, 16)] - zi
                d2 = dx * dx + dy * dy + dz * dz

                def merge():
                    cand_desc = lax.rev(jnp.sort(d2), (0,))
                    return jnp.sort(jnp.minimum(best, cand_desc))

                return lax.cond(jnp.any(d2 < best[15]), merge, lambda: best)

            best = lax.fori_loop(
                0, NBLK, blk_body, jnp.full((16,), jnp.inf, jnp.float32))
            res[pl.ds(r * 16, 16)] = best
            return carry

        lax.fori_loop(0, ROWS_PER_WORKER, row_body, 0)
        pltpu.sync_copy(
            res, out_hbm.at[pl.ds(base_row * 16, ROWS_PER_WORKER * 16)])

    return topk_kernel(xs, ys, zs)


def _tc_loss(top, s_pad):
    def body(top_ref, s_ref, out_ref):
        d = jnp.sqrt(top_ref[:, 1:9])
        elen = jnp.mean(d, axis=1, keepdims=True)
        diff = s_ref[:] - elen
        sq = diff * diff
        rows = lax.broadcasted_iota(jnp.int32, (NPAD, 1), 0)
        sq = jnp.where(rows < N, sq, 0.0)
        out_ref[0, 0] = jnp.sum(sq) / N

    return pl.pallas_call(
        body,
        out_shape=jax.ShapeDtypeStruct((1, 1), jnp.float32),
        out_specs=pl.BlockSpec(memory_space=pltpu.SMEM),
    )(top, s_pad)


def kernel(xyz_canon, scales):
    pad = jnp.full((COORD_LEN - N,), PAD_COORD, jnp.float32)
    xs = jnp.concatenate([xyz_canon[:, 0], pad])
    ys = jnp.concatenate([xyz_canon[:, 1], pad])
    zs = jnp.concatenate([xyz_canon[:, 2], pad])
    s_pad = jnp.concatenate(
        [scales[:, 0], jnp.zeros((NPAD - N,), jnp.float32)])[:, None]
    top = _sc_topk(xs, ys, zs).reshape(NPAD, 16)
    loss = _tc_loss(top, s_pad)
    return loss[0, 0]


# speedup vs baseline: 38.2294x; 7.0284x over previous
import functools

import jax
import jax.numpy as jnp
from jax import lax
from jax.experimental import pallas as pl
from jax.experimental.pallas import tpu as pltpu
from jax.experimental.pallas import tpu_sc as plsc

N = 10000
NUM_WORKERS = 32
ROWS_PER_WORKER = 313
NPAD = NUM_WORKERS * ROWS_PER_WORKER
NBLK = N // 16
COORD_LEN = NPAD + 16
PAD_COORD = 1.0e5


def _sc_topk(xs, ys, zs):
    mesh = plsc.VectorSubcoreMesh(
        core_axis_name="c", subcore_axis_name="s",
        num_cores=2, num_subcores=16,
    )

    @functools.partial(
        pl.kernel,
        out_type=jax.ShapeDtypeStruct((NPAD * 16,), jnp.float32),
        mesh=mesh,
        scratch_types=[
            pltpu.VMEM((COORD_LEN,), jnp.float32),
            pltpu.VMEM((COORD_LEN,), jnp.float32),
            pltpu.VMEM((COORD_LEN,), jnp.float32),
            pltpu.VMEM((ROWS_PER_WORKER * 16,), jnp.float32),
        ],
        compiler_params=pltpu.CompilerParams(needs_layout_passes=False),
    )
    def topk_kernel(xs_hbm, ys_hbm, zs_hbm, out_hbm, xv, yv, zv, res):
        wid = lax.axis_index("s") * 2 + lax.axis_index("c")
        pltpu.sync_copy(xs_hbm, xv)
        pltpu.sync_copy(ys_hbm, yv)
        pltpu.sync_copy(zs_hbm, zv)
        base_row = wid * ROWS_PER_WORKER

        def row_body(r, carry):
            row = base_row + r
            xi = xv[pl.ds(row, 16)][0]
            yi = yv[pl.ds(row, 16)][0]
            zi = zv[pl.ds(row, 16)][0]

            def blk_body(b, lane_best):
                off = b * 16
                dx = xv[pl.ds(off, 16)] - xi
                dy = yv[pl.ds(off, 16)] - yi
                dz = zv[pl.ds(off, 16)] - zi
                c = dx * dx + dy * dy + dz * dz
                out = []
                for bk in lane_best:
                    out.append(jnp.minimum(bk, c))
                    c = jnp.maximum(bk, c)
                return tuple(out)

            inf16 = jnp.full((16,), jnp.inf, jnp.float32)
            lane_best = lax.fori_loop(
                0, NBLK, blk_body, (inf16,) * 9, unroll=4)

            best = inf16
            for bk in lane_best:
                best = jnp.sort(jnp.minimum(best, lax.rev(jnp.sort(bk), (0,))))
            res[pl.ds(r * 16, 16)] = best
            return carry

        lax.fori_loop(0, ROWS_PER_WORKER, row_body, 0)
        pltpu.sync_copy(
            res, out_hbm.at[pl.ds(base_row * 16, ROWS_PER_WORKER * 16)])

    return topk_kernel(xs, ys, zs)


def _tc_loss(top, s_pad):
    def body(top_ref, s_ref, out_ref):
        d = jnp.sqrt(top_ref[:, 1:9])
        elen = jnp.mean(d, axis=1, keepdims=True)
        diff = s_ref[:] - elen
        sq = diff * diff
        rows = lax.broadcasted_iota(jnp.int32, (NPAD, 1), 0)
        sq = jnp.where(rows < N, sq, 0.0)
        out_ref[0, 0] = jnp.sum(sq) / N

    return pl.pallas_call(
        body,
        out_shape=jax.ShapeDtypeStruct((1, 1), jnp.float32),
        out_specs=pl.BlockSpec(memory_space=pltpu.SMEM),
    )(top, s_pad)


def kernel(xyz_canon, scales):
    pad = jnp.full((COORD_LEN - N,), PAD_COORD, jnp.float32)
    xs = jnp.concatenate([xyz_canon[:, 0], pad])
    ys = jnp.concatenate([xyz_canon[:, 1], pad])
    zs = jnp.concatenate([xyz_canon[:, 2], pad])
    s_pad = jnp.concatenate(
        [scales[:, 0], jnp.zeros((NPAD - N,), jnp.float32)])[:, None]
    top = _sc_topk(xs, ys, zs).reshape(NPAD, 16)
    loss = _tc_loss(top, s_pad)
    return loss[0, 0]


# speedup vs baseline: 46.4496x; 1.2150x over previous
import functools

import jax
import jax.numpy as jnp
from jax import lax
from jax.experimental import pallas as pl
from jax.experimental.pallas import tpu as pltpu
from jax.experimental.pallas import tpu_sc as plsc

N = 10000
NUM_WORKERS = 32
ROWS_PER_WORKER = 313
NPAD = NUM_WORKERS * ROWS_PER_WORKER
GROUP = 4
PAD_BLKS = 4
PADL = PAD_BLKS * 16
NBLK_TOT = PAD_BLKS + (N // 16) + PAD_BLKS
RIGHT_MAX = NBLK_TOT - GROUP
COORD_LEN = NBLK_TOT * 16 + 16
PAD_COORD = 1.0e5


def _sc_topk(xs, ys, zs):
    mesh = plsc.VectorSubcoreMesh(
        core_axis_name="c", subcore_axis_name="s",
        num_cores=2, num_subcores=16,
    )

    @functools.partial(
        pl.kernel,
        out_type=jax.ShapeDtypeStruct((NPAD * 16,), jnp.float32),
        mesh=mesh,
        scratch_types=[
            pltpu.VMEM((COORD_LEN,), jnp.float32),
            pltpu.VMEM((COORD_LEN,), jnp.float32),
            pltpu.VMEM((COORD_LEN,), jnp.float32),
            pltpu.VMEM((ROWS_PER_WORKER * 16,), jnp.float32),
        ],
        compiler_params=pltpu.CompilerParams(needs_layout_passes=False),
    )
    def topk_kernel(xs_hbm, ys_hbm, zs_hbm, out_hbm, xv, yv, zv, res):
        wid = lax.axis_index("s") * 2 + lax.axis_index("c")
        pltpu.sync_copy(xs_hbm, xv)
        pltpu.sync_copy(ys_hbm, yv)
        pltpu.sync_copy(zs_hbm, zv)

        inf16 = jnp.full((16,), jnp.inf, jnp.float32)

        def row_body(k, carry):
            srow = jnp.minimum(wid + 32 * k, N - 1)
            crow = srow + PADL
            xi = xv[pl.ds(crow, 16)][0]
            yi = yv[pl.ds(crow, 16)][0]
            zi = zv[pl.ds(crow, 16)][0]
            own = crow // 16

            def proc_group(p, best):
                for j in range(GROUP):
                    off = (p + j) * 16
                    dx = xv[pl.ds(off, 16)] - xi
                    dy = yv[pl.ds(off, 16)] - yi
                    dz = zv[pl.ds(off, 16)] - zi
                    c = dx * dx + dy * dy + dz * dz
                    nxt = []
                    for bk in best:
                        nxt.append(jnp.minimum(bk, c))
                        c = jnp.maximum(bk, c)
                    best = nxt
                return best

            def rcond(st):
                p, m = st[0], st[1]
                edge = xv[pl.ds(p * 16, 16)][0] - xi
                return (p <= RIGHT_MAX) & (edge * edge <= m)

            def rbody(st):
                p = st[0]
                best = proc_group(p, list(st[2:]))
                return (p + GROUP, jnp.min(best[8])) + tuple(best)

            st = lax.while_loop(
                rcond, rbody, (own, jnp.inf) + (inf16,) * 9)
            m_cur = st[1]
            best0 = st[2:]

            def lcond(st):
                p, m = st[0], st[1]
                edge = xv[pl.ds(jnp.maximum(p * 16 + 63, 0), 16)][0] - xi
                return (p >= 0) & (edge * edge <= m)

            def lbody(st):
                p = st[0]
                best = proc_group(p, list(st[2:]))
                return (p - GROUP, jnp.min(best[8])) + tuple(best)

            st = lax.while_loop(
                lcond, lbody, (own - GROUP, m_cur) + tuple(best0))
            lane_best = st[2:]

            best = inf16
            for bk in lane_best:
                best = jnp.sort(jnp.minimum(best, lax.rev(jnp.sort(bk), (0,))))
            res[pl.ds(k * 16, 16)] = best
            return carry

        lax.fori_loop(0, ROWS_PER_WORKER, row_body, 0)
        pltpu.sync_copy(
            res,
            out_hbm.at[pl.ds(wid * (ROWS_PER_WORKER * 16),
                             ROWS_PER_WORKER * 16)])

    return topk_kernel(xs, ys, zs)


def _tc_loss(top, s_aligned):
    def body(top_ref, s_ref, out_ref):
        d = jnp.sqrt(top_ref[:, 1:9])
        elen = jnp.mean(d, axis=1, keepdims=True)
        diff = s_ref[:] - elen
        sq = diff * diff
        t = lax.broadcasted_iota(jnp.int32, (NPAD, 1), 0)
        srow = t // ROWS_PER_WORKER + 32 * (t % ROWS_PER_WORKER)
        sq = jnp.where(srow < N, sq, 0.0)
        out_ref[0, 0] = jnp.sum(sq) / N

    return pl.pallas_call(
        body,
        out_shape=jax.ShapeDtypeStruct((1, 1), jnp.float32),
        out_specs=pl.BlockSpec(memory_space=pltpu.SMEM),
    )(top, s_aligned)


def kernel(xyz_canon, scales):
    order = jnp.argsort(xyz_canon[:, 0])
    xyz_s = xyz_canon[order]
    s_sorted = scales[order, 0]

    padl = jnp.full((PADL,), -PAD_COORD, jnp.float32)
    padr = jnp.full((COORD_LEN - PADL - N,), PAD_COORD, jnp.float32)
    xs = jnp.concatenate([padl, xyz_s[:, 0], padr])
    ys = jnp.concatenate([padl, xyz_s[:, 1], padr])
    zs = jnp.concatenate([padl, xyz_s[:, 2], padr])

    t = jnp.arange(NPAD, dtype=jnp.int32)
    srow = t // ROWS_PER_WORKER + 32 * (t % ROWS_PER_WORKER)
    s_aligned = jnp.where(
        srow < N, s_sorted[jnp.minimum(srow, N - 1)], 0.0)[:, None]

    top = _sc_topk(xs, ys, zs).reshape(NPAD, 16)
    loss = _tc_loss(top, s_aligned)
    return loss[0, 0]


# speedup vs baseline: 92.6691x; 1.9950x over previous
import functools

import jax
import jax.numpy as jnp
from jax import lax
from jax.experimental import pallas as pl
from jax.experimental.pallas import tpu as pltpu
from jax.experimental.pallas import tpu_sc as plsc

N = 10000
NUM_WORKERS = 32
ROWS_PER_WORKER = 313
NPAD = NUM_WORKERS * ROWS_PER_WORKER
GROUP = 8
PAD_BLKS = 8
PADL = PAD_BLKS * 16
NBLK_TOT = PAD_BLKS + (N // 16) + PAD_BLKS
RIGHT_MAX = NBLK_TOT - GROUP
COORD_LEN = NBLK_TOT * 16 + 16
PAD_COORD = 1.0e5


def _sc_topk(xs, ys, zs):
    mesh = plsc.VectorSubcoreMesh(
        core_axis_name="c", subcore_axis_name="s",
        num_cores=2, num_subcores=16,
    )

    @functools.partial(
        pl.kernel,
        out_type=jax.ShapeDtypeStruct((NPAD * 16,), jnp.float32),
        mesh=mesh,
        scratch_types=[
            pltpu.VMEM((COORD_LEN,), jnp.float32),
            pltpu.VMEM((COORD_LEN,), jnp.float32),
            pltpu.VMEM((COORD_LEN,), jnp.float32),
            pltpu.VMEM((ROWS_PER_WORKER * 16,), jnp.float32),
        ],
        compiler_params=pltpu.CompilerParams(needs_layout_passes=False),
    )
    def topk_kernel(xs_hbm, ys_hbm, zs_hbm, out_hbm, xv, yv, zv, res):
        wid = lax.axis_index("s") * 2 + lax.axis_index("c")
        pltpu.sync_copy(xs_hbm, xv)
        pltpu.sync_copy(ys_hbm, yv)
        pltpu.sync_copy(zs_hbm, zv)

        inf16 = jnp.full((16,), jnp.inf, jnp.float32)

        def row_body(k, carry):
            srow = jnp.minimum(wid + 32 * k, N - 1)
            crow = srow + PADL
            xi = xv[pl.ds(crow, 16)][0]
            yi = yv[pl.ds(crow, 16)][0]
            zi = zv[pl.ds(crow, 16)][0]
            own = crow // 16

            def proc_group(p, best):
                for j in range(GROUP):
                    off = (p + j) * 16
                    dx = xv[pl.ds(off, 16)] - xi
                    dy = yv[pl.ds(off, 16)] - yi
                    dz = zv[pl.ds(off, 16)] - zi
                    c = dx * dx + dy * dy + dz * dz
                    nxt = []
                    for bk in best:
                        nxt.append(jnp.minimum(bk, c))
                        c = jnp.maximum(bk, c)
                    best = nxt
                return best

            def rcond(st):
                p, m = st[0], st[1]
                edge = xv[pl.ds(p * 16, 16)][0] - xi
                return (p <= RIGHT_MAX) & (edge * edge <= m)

            def bound(m_prev, best):
                lo16 = jnp.sort(jnp.minimum(
                    jnp.sort(best[0]), lax.rev(jnp.sort(best[1]), (0,))))
                return jnp.minimum(
                    m_prev, jnp.minimum(jnp.min(best[8]), lo16[8]))

            def rbody(st):
                p = st[0]
                best = proc_group(p, list(st[2:]))
                return (p + GROUP, bound(st[1], best)) + tuple(best)

            st = lax.while_loop(
                rcond, rbody, (own, jnp.inf) + (inf16,) * 9)
            m_cur = st[1]
            best0 = st[2:]

            def lcond(st):
                p, m = st[0], st[1]
                edge = xv[pl.ds(jnp.maximum(p * 16 + (GROUP * 16 - 1), 0),
                                16)][0] - xi
                return (p >= 0) & (edge * edge <= m)

            def lbody(st):
                p = st[0]
                best = proc_group(p, list(st[2:]))
                return (p - GROUP, bound(st[1], best)) + tuple(best)

            st = lax.while_loop(
                lcond, lbody, (own - GROUP, m_cur) + tuple(best0))
            lane_best = st[2:]

            best = inf16
            for bk in lane_best:
                best = jnp.sort(jnp.minimum(best, lax.rev(jnp.sort(bk), (0,))))
            res[pl.ds(k * 16, 16)] = best
            return carry

        lax.fori_loop(0, ROWS_PER_WORKER, row_body, 0)
        pltpu.sync_copy(
            res,
            out_hbm.at[pl.ds(wid * (ROWS_PER_WORKER * 16),
                             ROWS_PER_WORKER * 16)])

    return topk_kernel(xs, ys, zs)


def _tc_loss(top, s_aligned):
    def body(top_ref, s_ref, out_ref):
        d = jnp.sqrt(top_ref[:, 1:9])
        elen = jnp.mean(d, axis=1, keepdims=True)
        diff = s_ref[:] - elen
        sq = diff * diff
        t = lax.broadcasted_iota(jnp.int32, (NPAD, 1), 0)
        srow = t // ROWS_PER_WORKER + 32 * (t % ROWS_PER_WORKER)
        sq = jnp.where(srow < N, sq, 0.0)
        out_ref[0, 0] = jnp.sum(sq) / N

    return pl.pallas_call(
        body,
        out_shape=jax.ShapeDtypeStruct((1, 1), jnp.float32),
        out_specs=pl.BlockSpec(memory_space=pltpu.SMEM),
    )(top, s_aligned)


def kernel(xyz_canon, scales):
    order = jnp.argsort(xyz_canon[:, 0])
    xyz_s = xyz_canon[order]
    s_sorted = scales[order, 0]

    padl = jnp.full((PADL,), -PAD_COORD, jnp.float32)
    padr = jnp.full((COORD_LEN - PADL - N,), PAD_COORD, jnp.float32)
    xs = jnp.concatenate([padl, xyz_s[:, 0], padr])
    ys = jnp.concatenate([padl, xyz_s[:, 1], padr])
    zs = jnp.concatenate([padl, xyz_s[:, 2], padr])

    t = jnp.arange(NPAD, dtype=jnp.int32)
    srow = t // ROWS_PER_WORKER + 32 * (t % ROWS_PER_WORKER)
    s_aligned = jnp.where(
        srow < N, s_sorted[jnp.minimum(srow, N - 1)], 0.0)[:, None]

    top = _sc_topk(xs, ys, zs).reshape(NPAD, 16)
    loss = _tc_loss(top, s_aligned)
    return loss[0, 0]


# speedup vs baseline: 97.3781x; 1.0508x over previous
import functools

import jax
import jax.numpy as jnp
from jax import lax
from jax.experimental import pallas as pl
from jax.experimental.pallas import tpu as pltpu
from jax.experimental.pallas import tpu_sc as plsc

N = 10000
NUM_WORKERS = 32
ROWS_PER_WORKER = 313
NPAD = NUM_WORKERS * ROWS_PER_WORKER
DEPTH = 8
GROUP = 8
PAD_BLKS = 8
PADL = PAD_BLKS * 16
NBLK_TOT = PAD_BLKS + (N // 16) + PAD_BLKS
RIGHT_MAX = NBLK_TOT - GROUP
COORD_LEN = NBLK_TOT * 16 + 16
PAD_COORD = 1.0e5
ROW_OUT = DEPTH * 16


def _sc_topk(xs, ys, zs):
    mesh = plsc.VectorSubcoreMesh(
        core_axis_name="c", subcore_axis_name="s",
        num_cores=2, num_subcores=16,
    )

    @functools.partial(
        pl.kernel,
        out_type=jax.ShapeDtypeStruct((NPAD * ROW_OUT,), jnp.float32),
        mesh=mesh,
        scratch_types=[
            pltpu.VMEM((COORD_LEN,), jnp.float32),
            pltpu.VMEM((COORD_LEN,), jnp.float32),
            pltpu.VMEM((COORD_LEN,), jnp.float32),
            pltpu.VMEM((ROWS_PER_WORKER * ROW_OUT,), jnp.float32),
        ],
        compiler_params=pltpu.CompilerParams(needs_layout_passes=False),
    )
    def topk_kernel(xs_hbm, ys_hbm, zs_hbm, out_hbm, xv, yv, zv, res):
        wid = lax.axis_index("s") * 2 + lax.axis_index("c")
        pltpu.sync_copy(xs_hbm, xv)
        pltpu.sync_copy(ys_hbm, yv)
        pltpu.sync_copy(zs_hbm, zv)

        inf16 = jnp.full((16,), jnp.inf, jnp.float32)
        lane0 = lax.iota(jnp.int32, 16) == 0

        def row_body(k, carry):
            srow = jnp.minimum(wid + 32 * k, N - 1)
            crow = srow + PADL
            w0 = xv[pl.ds(crow, 16)]
            xi = w0[0]
            yi = yv[pl.ds(crow, 16)][0]
            zi = zv[pl.ds(crow, 16)][0]
            own = crow // 16
            xv[pl.ds(crow, 16)] = jnp.where(lane0, PAD_COORD, w0)

            def proc_group(p, best):
                for j in range(GROUP):
                    off = (p + j) * 16
                    dx = xv[pl.ds(off, 16)] - xi
                    dy = yv[pl.ds(off, 16)] - yi
                    dz = zv[pl.ds(off, 16)] - zi
                    c = dx * dx + dy * dy + dz * dz
                    nxt = []
                    for bk in best:
                        nxt.append(jnp.minimum(bk, c))
                        c = jnp.maximum(bk, c)
                    best = nxt
                return best

            def bound(m_prev, best):
                lo16 = jnp.sort(jnp.minimum(
                    jnp.sort(best[0]), lax.rev(jnp.sort(best[1]), (0,))))
                return jnp.minimum(
                    m_prev, jnp.minimum(jnp.min(best[DEPTH - 1]), lo16[8]))

            def rcond(st):
                p, m = st[0], st[1]
                edge = xv[pl.ds(p * 16, 16)][0] - xi
                return (p <= RIGHT_MAX) & (edge * edge <= m)

            def rbody(st):
                p = st[0]
                best = proc_group(p, list(st[2:]))
                return (p + GROUP, bound(st[1], best)) + tuple(best)

            st = lax.while_loop(
                rcond, rbody, (own, jnp.inf) + (inf16,) * DEPTH)
            m_cur = st[1]
            best0 = st[2:]

            def lcond(st):
                p, m = st[0], st[1]
                edge = xv[pl.ds(jnp.maximum(p * 16 + (GROUP * 16 - 1), 0),
                                16)][0] - xi
                return (p >= 0) & (edge * edge <= m)

            def lbody(st):
                p = st[0]
                best = proc_group(p, list(st[2:]))
                return (p - GROUP, bound(st[1], best)) + tuple(best)

            st = lax.while_loop(
                lcond, lbody, (own - GROUP, m_cur) + tuple(best0))

            xv[pl.ds(crow, 16)] = w0
            for j in range(DEPTH):
                res[pl.ds(k * ROW_OUT + j * 16, 16)] = st[2 + j]
            return carry

        lax.fori_loop(0, ROWS_PER_WORKER, row_body, 0)
        pltpu.sync_copy(
            res,
            out_hbm.at[pl.ds(wid * (ROWS_PER_WORKER * ROW_OUT),
                             ROWS_PER_WORKER * ROW_OUT)])

    return topk_kernel(xs, ys, zs)


def _tc_loss(top, s_aligned):
    def body(top_ref, s_ref, out_ref):
        d = top_ref[:]
        total = jnp.zeros((NPAD, 1), jnp.float32)
        for _ in range(DEPTH):
            m = jnp.min(d, axis=1, keepdims=True)
            total = total + jnp.sqrt(m)
            d = jnp.where(d == m, jnp.inf, d)
        elen = total * (1.0 / DEPTH)
        diff = s_ref[:] - elen
        sq = diff * diff
        t = lax.broadcasted_iota(jnp.int32, (NPAD, 1), 0)
        srow = t // ROWS_PER_WORKER + 32 * (t % ROWS_PER_WORKER)
        sq = jnp.where(srow < N, sq, 0.0)
        out_ref[0, 0] = jnp.sum(sq) / N

    return pl.pallas_call(
        body,
        out_shape=jax.ShapeDtypeStruct((1, 1), jnp.float32),
        out_specs=pl.BlockSpec(memory_space=pltpu.SMEM),
    )(top, s_aligned)


def kernel(xyz_canon, scales):
    order = jnp.argsort(xyz_canon[:, 0])
    xyz_s = xyz_canon[order]
    s_sorted = scales[order, 0]

    padl = jnp.full((PADL,), -PAD_COORD, jnp.float32)
    padr = jnp.full((COORD_LEN - PADL - N,), PAD_COORD, jnp.float32)
    xs = jnp.concatenate([padl, xyz_s[:, 0], padr])
    ys = jnp.concatenate([padl, xyz_s[:, 1], padr])
    zs = jnp.concatenate([padl, xyz_s[:, 2], padr])

    t = jnp.arange(NPAD, dtype=jnp.int32)
    srow = t // ROWS_PER_WORKER + 32 * (t % ROWS_PER_WORKER)
    s_aligned = jnp.where(
        srow < N, s_sorted[jnp.minimum(srow, N - 1)], 0.0)[:, None]

    top = _sc_topk(xs, ys, zs).reshape(NPAD, ROW_OUT)
    loss = _tc_loss(top, s_aligned)
    return loss[0, 0]


# speedup vs baseline: 112.7948x; 1.1583x over previous
import functools

import jax
import jax.numpy as jnp
from jax import lax
from jax.experimental import pallas as pl
from jax.experimental.pallas import tpu as pltpu
from jax.experimental.pallas import tpu_sc as plsc

N = 10000
NUM_WORKERS = 32
ROWS_PER_WORKER = 313
NPAD = NUM_WORKERS * ROWS_PER_WORKER
DEPTH = 8
GROUP = 8
PAD_BLKS = 8
PADL = PAD_BLKS * 16
NBLK_TOT = PAD_BLKS + (N // 16) + PAD_BLKS
RIGHT_MAX = NBLK_TOT - GROUP
COORD_LEN = NBLK_TOT * 16 + 16
PAD_COORD = 1.0e5
ROW_OUT = DEPTH * 16


def _sc_topk(xs, ys, zs):
    mesh = plsc.VectorSubcoreMesh(
        core_axis_name="c", subcore_axis_name="s",
        num_cores=2, num_subcores=16,
    )

    @functools.partial(
        pl.kernel,
        out_type=jax.ShapeDtypeStruct(
            (ROWS_PER_WORKER, NUM_WORKERS, 1, ROW_OUT), jnp.float32),
        mesh=mesh,
        scratch_types=[
            pltpu.VMEM((COORD_LEN,), jnp.float32),
            pltpu.VMEM((COORD_LEN,), jnp.float32),
            pltpu.VMEM((COORD_LEN,), jnp.float32),
            pltpu.VMEM((ROWS_PER_WORKER, 1, ROW_OUT), jnp.float32),
        ],
        compiler_params=pltpu.CompilerParams(needs_layout_passes=False),
    )
    def topk_kernel(xs_hbm, ys_hbm, zs_hbm, out_hbm, xv, yv, zv, res):
        wid = lax.axis_index("s") * 2 + lax.axis_index("c")
        pltpu.sync_copy(xs_hbm, xv)
        pltpu.sync_copy(ys_hbm, yv)
        pltpu.sync_copy(zs_hbm, zv)

        inf16 = jnp.full((16,), jnp.inf, jnp.float32)
        lane0 = lax.iota(jnp.int32, 16) == 0

        def row_body(k, carry):
            srow = jnp.minimum(wid + 32 * k, N - 1)
            crow = srow + PADL
            w0 = xv[pl.ds(crow, 16)]
            xi = w0[0]
            yi = yv[pl.ds(crow, 16)][0]
            zi = zv[pl.ds(crow, 16)][0]
            own = crow // 16
            xv[pl.ds(crow, 16)] = jnp.where(lane0, PAD_COORD, w0)

            def proc_group(p, best):
                for j in range(GROUP):
                    off = (p + j) * 16
                    dx = xv[pl.ds(off, 16)] - xi
                    dy = yv[pl.ds(off, 16)] - yi
                    dz = zv[pl.ds(off, 16)] - zi
                    c = dx * dx + dy * dy + dz * dz
                    nxt = []
                    for bk in best:
                        nxt.append(jnp.minimum(bk, c))
                        c = jnp.maximum(bk, c)
                    best = nxt
                return best

            def bound(m_prev, best):
                lo16 = jnp.sort(jnp.minimum(
                    jnp.sort(best[0]), lax.rev(jnp.sort(best[1]), (0,))))
                return jnp.minimum(
                    m_prev, jnp.minimum(jnp.min(best[DEPTH - 1]), lo16[8]))

            def rcond(st):
                p, m = st[0], st[1]
                edge = xv[pl.ds(p * 16, 16)][0] - xi
                return (p <= RIGHT_MAX) & (edge * edge <= m)

            def rbody(st):
                p = st[0]
                best = proc_group(p, list(st[2:]))
                return (p + GROUP, bound(st[1], best)) + tuple(best)

            st = lax.while_loop(
                rcond, rbody, (own, jnp.inf) + (inf16,) * DEPTH)
            m_cur = st[1]
            best0 = st[2:]

            def lcond(st):
                p, m = st[0], st[1]
                edge = xv[pl.ds(jnp.maximum(p * 16 + (GROUP * 16 - 1), 0),
                                16)][0] - xi
                return (p >= 0) & (edge * edge <= m)

            def lbody(st):
                p = st[0]
                best = proc_group(p, list(st[2:]))
                return (p - GROUP, bound(st[1], best)) + tuple(best)

            st = lax.while_loop(
                lcond, lbody, (own - GROUP, m_cur) + tuple(best0))

            xv[pl.ds(crow, 16)] = w0
            for j in range(DEPTH):
                res[k, 0, pl.ds(j * 16, 16)] = st[2 + j]
            return carry

        lax.fori_loop(0, ROWS_PER_WORKER, row_body, 0)
        pltpu.sync_copy(res, out_hbm.at[:, wid])

    return topk_kernel(xs, ys, zs)


def _tc_loss(top, s_aligned):
    def body(top_ref, s_ref, out_ref):
        d = top_ref[:]
        total = jnp.zeros((NPAD, 1), jnp.float32)
        for _ in range(DEPTH):
            m = jnp.min(d, axis=1, keepdims=True)
            total = total + jnp.sqrt(m)
            d = jnp.where(d == m, jnp.inf, d)
        elen = total * (1.0 / DEPTH)
        diff = s_ref[:] - elen
        sq = diff * diff
        t = lax.broadcasted_iota(jnp.int32, (NPAD, 1), 0)
        sq = jnp.where(t < N, sq, 0.0)
        out_ref[0, 0] = jnp.sum(sq) / N

    return pl.pallas_call(
        body,
        out_shape=jax.ShapeDtypeStruct((1, 1), jnp.float32),
        out_specs=pl.BlockSpec(memory_space=pltpu.SMEM),
    )(top, s_aligned)


def kernel(xyz_canon, scales):
    xc, yc, zc, s_sorted = lax.sort(
        (xyz_canon[:, 0], xyz_canon[:, 1], xyz_canon[:, 2], scales[:, 0]),
        num_keys=1)

    padl = jnp.full((PADL,), -PAD_COORD, jnp.float32)
    padr = jnp.full((COORD_LEN - PADL - N,), PAD_COORD, jnp.float32)
    xs = jnp.concatenate([padl, xc, padr])
    ys = jnp.concatenate([padl, yc, padr])
    zs = jnp.concatenate([padl, zc, padr])
    s_pad = jnp.concatenate(
        [s_sorted, jnp.zeros((NPAD - N,), jnp.float32)])[:, None]

    top = _sc_topk(xs, ys, zs).reshape(NPAD, ROW_OUT)
    loss = _tc_loss(top, s_pad)
    return loss[0, 0]


# speedup vs baseline: 128.1882x; 1.1365x over previous
import functools

import jax
import jax.numpy as jnp
from jax import lax
from jax.experimental import pallas as pl
from jax.experimental.pallas import tpu as pltpu
from jax.experimental.pallas import tpu_sc as plsc

N = 10000
NUM_WORKERS = 32
ROWS_PER_WORKER = 313
NPAD = NUM_WORKERS * ROWS_PER_WORKER
DEPTH = 8
GROUP = 16
PAD_BLKS = 16
PADL = PAD_BLKS * 16
NBLK_TOT = PAD_BLKS + (N // 16) + PAD_BLKS
RIGHT_MAX = NBLK_TOT - GROUP
COORD_LEN = NBLK_TOT * 16 + 16
PAD_COORD = 1.0e5
ROW_OUT = DEPTH * 16


def _sc_topk(xs, ys, zs):
    mesh = plsc.VectorSubcoreMesh(
        core_axis_name="c", subcore_axis_name="s",
        num_cores=2, num_subcores=16,
    )

    @functools.partial(
        pl.kernel,
        out_type=jax.ShapeDtypeStruct(
            (ROWS_PER_WORKER, NUM_WORKERS, 1, ROW_OUT), jnp.float32),
        mesh=mesh,
        scratch_types=[
            pltpu.VMEM((COORD_LEN,), jnp.float32),
            pltpu.VMEM((COORD_LEN,), jnp.float32),
            pltpu.VMEM((COORD_LEN,), jnp.float32),
            pltpu.VMEM((ROWS_PER_WORKER, 1, ROW_OUT), jnp.float32),
        ],
        compiler_params=pltpu.CompilerParams(needs_layout_passes=False),
    )
    def topk_kernel(xs_hbm, ys_hbm, zs_hbm, out_hbm, xv, yv, zv, res):
        wid = lax.axis_index("s") * 2 + lax.axis_index("c")
        pltpu.sync_copy(xs_hbm, xv)
        pltpu.sync_copy(ys_hbm, yv)
        pltpu.sync_copy(zs_hbm, zv)

        inf16 = jnp.full((16,), jnp.inf, jnp.float32)
        lane0 = lax.iota(jnp.int32, 16) == 0

        def row_body(k, carry):
            srow = jnp.minimum(wid + 32 * k, N - 1)
            crow = srow + PADL
            w0 = xv[pl.ds(crow, 16)]
            xi = w0[0]
            yi = yv[pl.ds(crow, 16)][0]
            zi = zv[pl.ds(crow, 16)][0]
            own = crow // 16
            xv[pl.ds(crow, 16)] = jnp.where(lane0, PAD_COORD, w0)

            def proc_group(p, best):
                for j in range(GROUP):
                    off = (p + j) * 16
                    dx = xv[pl.ds(off, 16)] - xi
                    dy = yv[pl.ds(off, 16)] - yi
                    dz = zv[pl.ds(off, 16)] - zi
                    c = dx * dx + dy * dy + dz * dz
                    nxt = []
                    for bk in best:
                        nxt.append(jnp.minimum(bk, c))
                        c = jnp.maximum(bk, c)
                    best = nxt
                return best

            def bound(m_prev, best):
                lo16 = jnp.sort(jnp.minimum(
                    jnp.sort(best[0]), lax.rev(jnp.sort(best[1]), (0,))))
                return jnp.minimum(
                    m_prev, jnp.minimum(jnp.min(best[DEPTH - 1]), lo16[8]))

            def rcond(st):
                p, m = st[0], st[1]
                edge = xv[pl.ds(p * 16, 16)][0] - xi
                return (p <= RIGHT_MAX) & (edge * edge <= m)

            def rbody(st):
                p = st[0]
                best = proc_group(p, list(st[2:]))
                return (p + GROUP, bound(st[1], best)) + tuple(best)

            st = lax.while_loop(
                rcond, rbody, (own, jnp.inf) + (inf16,) * DEPTH)
            m_cur = st[1]
            best0 = st[2:]

            def lcond(st):
                p, m = st[0], st[1]
                edge = xv[pl.ds(jnp.maximum(p * 16 + (GROUP * 16 - 1), 0),
                                16)][0] - xi
                return (p >= 0) & (edge * edge <= m)

            def lbody(st):
                p = st[0]
                best = proc_group(p, list(st[2:]))
                return (p - GROUP, bound(st[1], best)) + tuple(best)

            st = lax.while_loop(
                lcond, lbody, (own - GROUP, m_cur) + tuple(best0))

            xv[pl.ds(crow, 16)] = w0
            for j in range(DEPTH):
                res[k, 0, pl.ds(j * 16, 16)] = st[2 + j]
            return carry

        lax.fori_loop(0, ROWS_PER_WORKER, row_body, 0)
        pltpu.sync_copy(res, out_hbm.at[:, wid])

    return topk_kernel(xs, ys, zs)


def _tc_loss(top, s_aligned):
    def body(top_ref, s_ref, out_ref):
        d = top_ref[:]
        total = jnp.zeros((NPAD, 1), jnp.float32)
        for _ in range(DEPTH):
            m = jnp.min(d, axis=1, keepdims=True)
            total = total + jnp.sqrt(m)
            d = jnp.where(d == m, jnp.inf, d)
        elen = total * (1.0 / DEPTH)
        diff = s_ref[:] - elen
        sq = diff * diff
        t = lax.broadcasted_iota(jnp.int32, (NPAD, 1), 0)
        sq = jnp.where(t < N, sq, 0.0)
        out_ref[0, 0] = jnp.sum(sq) / N

    return pl.pallas_call(
        body,
        out_shape=jax.ShapeDtypeStruct((1, 1), jnp.float32),
        out_specs=pl.BlockSpec(memory_space=pltpu.SMEM),
    )(top, s_aligned)


def kernel(xyz_canon, scales):
    xc, yc, zc, s_sorted = lax.sort(
        (xyz_canon[:, 0], xyz_canon[:, 1], xyz_canon[:, 2], scales[:, 0]),
        num_keys=1)

    padl = jnp.full((PADL,), -PAD_COORD, jnp.float32)
    padr = jnp.full((COORD_LEN - PADL - N,), PAD_COORD, jnp.float32)
    xs = jnp.concatenate([padl, xc, padr])
    ys = jnp.concatenate([padl, yc, padr])
    zs = jnp.concatenate([padl, zc, padr])
    s_pad = jnp.concatenate(
        [s_sorted, jnp.zeros((NPAD - N,), jnp.float32)])[:, None]

    top = _sc_topk(xs, ys, zs).reshape(NPAD, ROW_OUT)
    loss = _tc_loss(top, s_pad)
    return loss[0, 0]


# speedup vs baseline: 132.9859x; 1.0374x over previous
import functools

import jax
import jax.numpy as jnp
from jax import lax
from jax.experimental import pallas as pl
from jax.experimental.pallas import tpu as pltpu
from jax.experimental.pallas import tpu_sc as plsc

N = 10000
NUM_WORKERS = 32
PAIRS_PER_WORKER = 157
NPAD = NUM_WORKERS * PAIRS_PER_WORKER * 2
DEPTH = 8
GROUP = 16
PAD_BLKS = 16
PADL = PAD_BLKS * 16
NBLK_TOT = PAD_BLKS + (N // 16) + PAD_BLKS
RIGHT_MAX = NBLK_TOT - GROUP
COORD_LEN = NBLK_TOT * 16 + 16
PAD_COORD = 1.0e5
ROW_OUT = DEPTH * 16


def _sc_topk(xs, ys, zs):
    mesh = plsc.VectorSubcoreMesh(
        core_axis_name="c", subcore_axis_name="s",
        num_cores=2, num_subcores=16,
    )

    @functools.partial(
        pl.kernel,
        out_type=jax.ShapeDtypeStruct(
            (PAIRS_PER_WORKER, NUM_WORKERS, 2, 1, ROW_OUT), jnp.float32),
        mesh=mesh,
        scratch_types=[
            pltpu.VMEM((COORD_LEN,), jnp.float32),
            pltpu.VMEM((COORD_LEN,), jnp.float32),
            pltpu.VMEM((COORD_LEN,), jnp.float32),
            pltpu.VMEM((PAIRS_PER_WORKER, 2, 1, ROW_OUT), jnp.float32),
        ],
        compiler_params=pltpu.CompilerParams(needs_layout_passes=False),
    )
    def topk_kernel(xs_hbm, ys_hbm, zs_hbm, out_hbm, xv, yv, zv, res):
        wid = lax.axis_index("s") * 2 + lax.axis_index("c")
        pltpu.sync_copy(xs_hbm, xv)
        pltpu.sync_copy(ys_hbm, yv)
        pltpu.sync_copy(zs_hbm, zv)

        inf16 = jnp.full((16,), jnp.inf, jnp.float32)
        lane01 = lax.iota(jnp.int32, 16) < 2

        def pair_body(k, carry):
            srow_a = jnp.minimum(2 * (wid + 32 * k), N - 2)
            crow = srow_a + PADL
            wx = xv[pl.ds(crow, 16)]
            wy = yv[pl.ds(crow, 16)]
            wz = zv[pl.ds(crow, 16)]
            xa, xb = wx[0], wx[1]
            ya, yb = wy[0], wy[1]
            za, zb = wz[0], wz[1]
            own = crow // 16
            xv[pl.ds(crow, 16)] = jnp.where(lane01, PAD_COORD, wx)

            def proc_group(p, best):
                ba, bb = best[:DEPTH], best[DEPTH:]
                for j in range(GROUP):
                    off = (p + j) * 16
                    cx = xv[pl.ds(off, 16)]
                    cy = yv[pl.ds(off, 16)]
                    cz = zv[pl.ds(off, 16)]
                    dxa = cx - xa
                    dya = cy - ya
                    dza = cz - za
                    ca = dxa * dxa + dya * dya + dza * dza
                    dxb = cx - xb
                    dyb = cy - yb
                    dzb = cz - zb
                    cb = dxb * dxb + dyb * dyb + dzb * dzb
                    na, nb = [], []
                    for bk in ba:
                        na.append(jnp.minimum(bk, ca))
                        ca = jnp.maximum(bk, ca)
                    for bk in bb:
                        nb.append(jnp.minimum(bk, cb))
                        cb = jnp.maximum(bk, cb)
                    ba, bb = na, nb
                return list(ba) + list(bb)

            def bound(m_prev, b0, b1, blast):
                lo16 = jnp.sort(jnp.minimum(
                    jnp.sort(b0), lax.rev(jnp.sort(b1), (0,))))
                return jnp.minimum(
                    m_prev, jnp.minimum(jnp.min(blast), lo16[8]))

            def bounds(st, best):
                ma = bound(st[1], best[0], best[1], best[DEPTH - 1])
                mb = bound(st[2], best[DEPTH], best[DEPTH + 1],
                           best[2 * DEPTH - 1])
                return ma, mb

            def rcond(st):
                p = st[0]
                edge = xv[pl.ds(p * 16, 16)][0] - xa
                return (p <= RIGHT_MAX) & (edge * edge <= jnp.maximum(
                    st[1], st[2]))

            def rbody(st):
                p = st[0]
                best = proc_group(p, list(st[3:]))
                ma, mb = bounds(st, best)
                return (p + GROUP, ma, mb) + tuple(best)

            st = lax.while_loop(
                rcond, rbody, (own, jnp.inf, jnp.inf) + (inf16,) * (2 * DEPTH))
            ma_cur, mb_cur = st[1], st[2]
            best0 = st[3:]

            def lcond(st):
                p = st[0]
                edge = xv[pl.ds(jnp.maximum(p * 16 + (GROUP * 16 - 1), 0),
                                16)][0] - xa
                return (p >= 0) & (edge * edge <= jnp.maximum(st[1], st[2]))

            def lbody(st):
                p = st[0]
                best = proc_group(p, list(st[3:]))
                ma, mb = bounds(st, best)
                return (p - GROUP, ma, mb) + tuple(best)

            st = lax.while_loop(
                lcond, lbody, (own - GROUP, ma_cur, mb_cur) + tuple(best0))

            xv[pl.ds(crow, 16)] = wx
            for j in range(DEPTH):
                res[k, 0, 0, pl.ds(j * 16, 16)] = st[3 + j]
                res[k, 1, 0, pl.ds(j * 16, 16)] = st[3 + DEPTH + j]
            return carry

        lax.fori_loop(0, PAIRS_PER_WORKER, pair_body, 0)
        pltpu.sync_copy(res, out_hbm.at[:, wid])

    return topk_kernel(xs, ys, zs)


def _tc_loss(top, s_aligned):
    def body(top_ref, s_ref, out_ref):
        d = top_ref[:]
        total = jnp.zeros((NPAD, 1), jnp.float32)
        for _ in range(DEPTH):
            m = jnp.min(d, axis=1, keepdims=True)
            total = total + jnp.sqrt(m)
            d = jnp.where(d == m, jnp.inf, d)
        elen = total * (1.0 / DEPTH)
        diff = s_ref[:] - elen
        sq = diff * diff
        t = lax.broadcasted_iota(jnp.int32, (NPAD, 1), 0)
        sq = jnp.where(t < N, sq, 0.0)
        out_ref[0, 0] = jnp.sum(sq) / N

    return pl.pallas_call(
        body,
        out_shape=jax.ShapeDtypeStruct((1, 1), jnp.float32),
        out_specs=pl.BlockSpec(memory_space=pltpu.SMEM),
    )(top, s_aligned)


def kernel(xyz_canon, scales):
    xc, yc, zc, s_sorted = lax.sort(
        (xyz_canon[:, 0], xyz_canon[:, 1], xyz_canon[:, 2], scales[:, 0]),
        num_keys=1)

    padl = jnp.full((PADL,), -PAD_COORD, jnp.float32)
    padr = jnp.full((COORD_LEN - PADL - N,), PAD_COORD, jnp.float32)
    xs = jnp.concatenate([padl, xc, padr])
    ys = jnp.concatenate([padl, yc, padr])
    zs = jnp.concatenate([padl, zc, padr])
    s_pad = jnp.concatenate(
        [s_sorted, jnp.zeros((NPAD - N,), jnp.float32)])[:, None]

    top = _sc_topk(xs, ys, zs).reshape(NPAD, ROW_OUT)
    loss = _tc_loss(top, s_pad)
    return loss[0, 0]


# speedup vs baseline: 144.6207x; 1.0875x over previous
import functools

import jax
import jax.numpy as jnp
from jax import lax
from jax.experimental import pallas as pl
from jax.experimental.pallas import tpu as pltpu
from jax.experimental.pallas import tpu_sc as plsc

N = 10000
NUM_WORKERS = 32
PAIRS_PER_WORKER = 157
NPAD = NUM_WORKERS * PAIRS_PER_WORKER * 2
DEPTH = 8
GROUP = 16
PAD_BLKS = 16
PADL = PAD_BLKS * 16
NBLK_TOT = PAD_BLKS + (N // 16) + PAD_BLKS
RIGHT_MAX = NBLK_TOT - GROUP
COORD_LEN = NBLK_TOT * 16 + 16
PAD_COORD = 1.0e5
ROW_OUT = DEPTH * 16


def _sc_topk(xs, ys, zs):
    mesh = plsc.VectorSubcoreMesh(
        core_axis_name="c", subcore_axis_name="s",
        num_cores=2, num_subcores=16,
    )

    @functools.partial(
        pl.kernel,
        out_type=jax.ShapeDtypeStruct(
            (PAIRS_PER_WORKER, NUM_WORKERS, 2, 1, ROW_OUT), jnp.float32),
        mesh=mesh,
        scratch_types=[
            pltpu.VMEM((COORD_LEN,), jnp.float32),
            pltpu.VMEM((COORD_LEN,), jnp.float32),
            pltpu.VMEM((COORD_LEN,), jnp.float32),
            pltpu.VMEM((PAIRS_PER_WORKER, 2, 1, ROW_OUT), jnp.float32),
        ],
        compiler_params=pltpu.CompilerParams(needs_layout_passes=False),
    )
    def topk_kernel(xs_hbm, ys_hbm, zs_hbm, out_hbm, xv, yv, zv, res):
        wid = lax.axis_index("s") * 2 + lax.axis_index("c")
        pltpu.sync_copy(xs_hbm, xv)
        pltpu.sync_copy(ys_hbm, yv)
        pltpu.sync_copy(zs_hbm, zv)

        inf16 = jnp.full((16,), jnp.inf, jnp.float32)
        lane01 = lax.iota(jnp.int32, 16) < 2

        def pair_body(k, carry):
            srow_a = jnp.minimum(2 * (wid + 32 * k), N - 2)
            crow = srow_a + PADL
            wx = xv[pl.ds(crow, 16)]
            wy = yv[pl.ds(crow, 16)]
            wz = zv[pl.ds(crow, 16)]
            xa, xb = wx[0], wx[1]
            ya, yb = wy[0], wy[1]
            za, zb = wz[0], wz[1]
            own = crow // 16
            xv[pl.ds(crow, 16)] = jnp.where(lane01, PAD_COORD, wx)

            def proc_group(p, best):
                ba, bb = best[:DEPTH], best[DEPTH:]
                for j in range(GROUP):
                    off = (p + j) * 16
                    cx = xv[pl.ds(off, 16)]
                    cy = yv[pl.ds(off, 16)]
                    cz = zv[pl.ds(off, 16)]
                    dxa = cx - xa
                    dya = cy - ya
                    dza = cz - za
                    ca = dxa * dxa + dya * dya + dza * dza
                    dxb = cx - xb
                    dyb = cy - yb
                    dzb = cz - zb
                    cb = dxb * dxb + dyb * dyb + dzb * dzb
                    na, nb = [], []
                    for bk in ba:
                        na.append(jnp.minimum(bk, ca))
                        ca = jnp.maximum(bk, ca)
                    for bk in bb:
                        nb.append(jnp.minimum(bk, cb))
                        cb = jnp.maximum(bk, cb)
                    ba, bb = na, nb
                return list(ba) + list(bb)

            def bound(m_prev, b0, b1):
                lo16 = jnp.sort(jnp.minimum(
                    jnp.sort(b0), lax.rev(jnp.sort(b1), (0,))))
                return jnp.minimum(m_prev, lo16[8])

            def bounds(st, best):
                ma = bound(st[1], best[0], best[1])
                mb = bound(st[2], best[DEPTH], best[DEPTH + 1])
                return ma, mb

            def rcond(st):
                p = st[0]
                edge = xv[pl.ds(p * 16, 16)][0] - xa
                return (p <= RIGHT_MAX) & (edge * edge <= jnp.maximum(
                    st[1], st[2]))

            def rbody(st):
                p = st[0]
                best = proc_group(p, list(st[3:]))
                ma, mb = bounds(st, best)
                return (p + GROUP, ma, mb) + tuple(best)

            st = lax.while_loop(
                rcond, rbody, (own, jnp.inf, jnp.inf) + (inf16,) * (2 * DEPTH))
            ma_cur, mb_cur = st[1], st[2]
            best0 = st[3:]

            def lcond(st):
                p = st[0]
                edge = xv[pl.ds(jnp.maximum(p * 16 + (GROUP * 16 - 1), 0),
                                16)][0] - xa
                return (p >= 0) & (edge * edge <= jnp.maximum(st[1], st[2]))

            def lbody(st):
                p = st[0]
                best = proc_group(p, list(st[3:]))
                ma, mb = bounds(st, best)
                return (p - GROUP, ma, mb) + tuple(best)

            st = lax.while_loop(
                lcond, lbody, (own - GROUP, ma_cur, mb_cur) + tuple(best0))

            xv[pl.ds(crow, 16)] = wx
            for j in range(DEPTH):
                res[k, 0, 0, pl.ds(j * 16, 16)] = st[3 + j]
                res[k, 1, 0, pl.ds(j * 16, 16)] = st[3 + DEPTH + j]
            return carry

        lax.fori_loop(0, PAIRS_PER_WORKER, pair_body, 0)
        pltpu.sync_copy(res, out_hbm.at[:, wid])

    return topk_kernel(xs, ys, zs)


def _tc_loss(top, s_aligned):
    def body(top_ref, s_ref, out_ref):
        d = top_ref[:]
        total = jnp.zeros((NPAD, 1), jnp.float32)
        for _ in range(DEPTH):
            m = jnp.min(d, axis=1, keepdims=True)
            total = total + jnp.sqrt(m)
            d = jnp.where(d == m, jnp.inf, d)
        elen = total * (1.0 / DEPTH)
        diff = s_ref[:] - elen
        sq = diff * diff
        t = lax.broadcasted_iota(jnp.int32, (NPAD, 1), 0)
        sq = jnp.where(t < N, sq, 0.0)
        out_ref[0, 0] = jnp.sum(sq) / N

    return pl.pallas_call(
        body,
        out_shape=jax.ShapeDtypeStruct((1, 1), jnp.float32),
        out_specs=pl.BlockSpec(memory_space=pltpu.SMEM),
    )(top, s_aligned)


def kernel(xyz_canon, scales):
    xc, yc, zc, s_sorted = lax.sort(
        (xyz_canon[:, 0], xyz_canon[:, 1], xyz_canon[:, 2], scales[:, 0]),
        num_keys=1)

    padl = jnp.full((PADL,), -PAD_COORD, jnp.float32)
    padr = jnp.full((COORD_LEN - PADL - N,), PAD_COORD, jnp.float32)
    xs = jnp.concatenate([padl, xc, padr])
    ys = jnp.concatenate([padl, yc, padr])
    zs = jnp.concatenate([padl, zc, padr])
    s_pad = jnp.concatenate(
        [s_sorted, jnp.zeros((NPAD - N,), jnp.float32)])[:, None]

    top = _sc_topk(xs, ys, zs).reshape(NPAD, ROW_OUT)
    loss = _tc_loss(top, s_pad)
    return loss[0, 0]
